# baseline (speedup 1.0000x reference)
import jax
import jax.numpy as jnp
from jax.experimental import pallas as pl
from jax.experimental.pallas import tpu as pltpu

_ROWS = 16384
_COLS = 256
_CHUNKS = 4
_CHUNK_ROWS = _ROWS // _CHUNKS


def _dma_body(t_hbm, q_hbm, t_out, q_out, sems):
    for i in range(_CHUNKS):
        sl = pl.ds(i * _CHUNK_ROWS, _CHUNK_ROWS)
        pltpu.make_async_copy(t_hbm.at[sl], t_out.at[sl], sems.at[2 * i]).start()
        pltpu.make_async_copy(q_hbm.at[sl], q_out.at[sl], sems.at[2 * i + 1]).start()
    for i in range(_CHUNKS):
        sl = pl.ds(i * _CHUNK_ROWS, _CHUNK_ROWS)
        pltpu.make_async_copy(t_hbm.at[sl], t_out.at[sl], sems.at[2 * i]).wait()
        pltpu.make_async_copy(q_hbm.at[sl], q_out.at[sl], sems.at[2 * i + 1]).wait()


def kernel(emb_targets, emb_queries):
    any_spec = pl.BlockSpec(memory_space=pl.MemorySpace.ANY)
    out_t, out_q = pl.pallas_call(
        _dma_body,
        in_specs=[any_spec, any_spec],
        out_specs=[any_spec, any_spec],
        out_shape=[
            jax.ShapeDtypeStruct((_ROWS, _COLS), jnp.float32),
            jax.ShapeDtypeStruct((_ROWS, _COLS), jnp.float32),
        ],
        scratch_shapes=[pltpu.SemaphoreType.DMA((2 * _CHUNKS,))],
    )(emb_targets, emb_queries)
    return (out_t, out_q)


# speedup vs baseline: 40.0361x; 40.0361x over previous
import jax
import jax.numpy as jnp
from jax.experimental import pallas as pl

_ROWS = 16384
_COLS = 256
_BLOCK_ROWS = 1024


def _copy_body(t_ref, q_ref, t_out, q_out):
    t_out[...] = t_ref[...]
    q_out[...] = q_ref[...]


def kernel(emb_targets, emb_queries):
    grid = (_ROWS // _BLOCK_ROWS,)
    spec = pl.BlockSpec((_BLOCK_ROWS, _COLS), lambda i: (i, 0))
    out_t, out_q = pl.pallas_call(
        _copy_body,
        grid=grid,
        in_specs=[spec, spec],
        out_specs=[spec, spec],
        out_shape=[
            jax.ShapeDtypeStruct((_ROWS, _COLS), jnp.float32),
            jax.ShapeDtypeStruct((_ROWS, _COLS), jnp.float32),
        ],
    )(emb_targets, emb_queries)
    return (out_t, out_q)


# speedup vs baseline: 46.3474x; 1.1576x over previous
import jax
import jax.numpy as jnp
from jax.experimental import pallas as pl

_ROWS = 16384
_COLS = 256
_BLOCK_ROWS = 4096


def _copy_body(t_ref, q_ref, t_out, q_out):
    t_out[...] = t_ref[...]
    q_out[...] = q_ref[...]


def kernel(emb_targets, emb_queries):
    grid = (_ROWS // _BLOCK_ROWS,)
    spec = pl.BlockSpec((_BLOCK_ROWS, _COLS), lambda i: (i, 0))
    out_t, out_q = pl.pallas_call(
        _copy_body,
        grid=grid,
        in_specs=[spec, spec],
        out_specs=[spec, spec],
        out_shape=[
            jax.ShapeDtypeStruct((_ROWS, _COLS), jnp.float32),
            jax.ShapeDtypeStruct((_ROWS, _COLS), jnp.float32),
        ],
    )(emb_targets, emb_queries)
    return (out_t, out_q)
